# baseline (speedup 1.0000x reference)
import jax
import jax.numpy as jnp
from jax.experimental import pallas as pl
from jax.experimental.pallas import tpu as pltpu


def _roll_copy(x_ref, cache_ref, out_ref, sems):
    B, S, D = out_ref.shape
    Q = x_ref.shape[1]
    copies = []
    for b in range(B):
        copies.append(pltpu.make_async_copy(
            cache_ref.at[b, pl.ds(Q, S - Q), :],
            out_ref.at[b, pl.ds(0, S - Q), :],
            sems.at[b]))
        copies.append(pltpu.make_async_copy(
            x_ref.at[b],
            out_ref.at[b, pl.ds(S - Q, Q), :],
            sems.at[B + b]))
    for c in copies:
        c.start()
    for c in copies:
        c.wait()


@jax.jit
def kernel(x, cache):
    B, S, D = cache.shape
    return pl.pallas_call(
        _roll_copy,
        out_shape=jax.ShapeDtypeStruct((B, S, D), cache.dtype),
        in_specs=[pl.BlockSpec(memory_space=pl.ANY),
                  pl.BlockSpec(memory_space=pl.ANY)],
        out_specs=pl.BlockSpec(memory_space=pl.ANY),
        scratch_shapes=[pltpu.SemaphoreType.DMA((2 * B,))],
    )(x, cache)
